# baseline (speedup 1.0000x reference)
import jax
import jax.numpy as jnp
from jax.experimental import pallas as pl
from jax.experimental.pallas import tpu as pltpu

_TILE_N = 4096
_INV_T = 20.0


def _mm_kernel(x_ref, m_ref, o_ref):
    x = (x_ref[...] * _INV_T).astype(jnp.bfloat16)
    m = m_ref[...].astype(jnp.bfloat16)
    o_ref[...] = jax.lax.dot_general(
        x, m, (((1,), (1,)), ((), ())),
        preferred_element_type=jnp.float32)


def kernel(x, mem_feat):
    q, k = x.shape
    n = mem_feat.shape[0]
    return pl.pallas_call(
        _mm_kernel,
        grid=(n // _TILE_N,),
        in_specs=[
            pl.BlockSpec((q, k), lambda i: (0, 0)),
            pl.BlockSpec((_TILE_N, k), lambda i: (i, 0)),
        ],
        out_specs=pl.BlockSpec((q, _TILE_N), lambda i: (0, i)),
        out_shape=jax.ShapeDtypeStruct((q, n), jnp.float32),
        compiler_params=pltpu.CompilerParams(
            dimension_semantics=("parallel",),
            vmem_limit_bytes=120 * 1024 * 1024),
    )(x, mem_feat)


# speedup vs baseline: 1.0021x; 1.0021x over previous
import jax
import jax.numpy as jnp
from jax.experimental import pallas as pl
from jax.experimental.pallas import tpu as pltpu

_TILE_N = 4096
_INV_T = 20.0


def _mm_kernel(x_ref, m_ref, o_ref):
    x = (x_ref[...] * _INV_T).astype(jnp.bfloat16)
    m = m_ref[...].astype(jnp.bfloat16)
    o_ref[...] = jax.lax.dot_general(
        x, m, (((1,), (1,)), ((), ())),
        preferred_element_type=jnp.float32)


def kernel(x, mem_feat):
    q, k = x.shape
    n = mem_feat.shape[0]
    return pl.pallas_call(
        _mm_kernel,
        grid=(n // _TILE_N,),
        in_specs=[
            pl.BlockSpec((q, k), lambda i: (0, 0)),
            pl.BlockSpec((_TILE_N, k), lambda i: (i, 0)),
        ],
        out_specs=pl.BlockSpec((q, _TILE_N), lambda i: (0, i)),
        out_shape=jax.ShapeDtypeStruct((q, n), jnp.float32),
    )(x, mem_feat)
